# baseline (speedup 1.0000x reference)
import numpy as np
import jax
import jax.numpy as jnp
from jax.experimental import pallas as pl
from jax.experimental.pallas import tpu as pltpu

_BT = 256


def _conv1_toeplitz_idx():
    idx = np.full((192, 480), 450, np.int32)
    for par in range(2):
        for co in range(6):
            for t in range(14):
                ow = 2 * t + par
                for ci in range(3):
                    for kh in range(5):
                        for kw in range(5):
                            idx[par * 96 + co * 16 + t,
                                ci * 160 + kh * 32 + ow + kw] = (
                                co * 75 + ci * 25 + kh * 5 + kw)
    return idx


def _conv2_toeplitz_idx():
    idx = np.full((256, 480), 2400, np.int32)
    for par in range(2):
        for co in range(16):
            for t in range(5):
                ow = 2 * t + par
                for ci in range(6):
                    for kh in range(5):
                        for kw in range(5):
                            idx[par * 128 + co * 8 + t,
                                ci * 80 + kh * 16 + ow + kw] = (
                                co * 150 + ci * 25 + kh * 5 + kw)
    return idx


_IDX1 = _conv1_toeplitz_idx()
_IDX2 = _conv2_toeplitz_idx()
_T1MASK = (np.arange(16) < 14).astype(np.float32)
_T2MASK = (np.arange(8) < 5).astype(np.float32)


def _lenet_kernel(xb_ref, wt1_ref, b1_ref, wt2_ref, b2_ref,
                  f1w_ref, f1b_ref, f2w_ref, f2b_ref, f3w_ref, f3b_ref,
                  out_ref,
                  p1_ref, act_ref):
    f32 = jnp.float32
    bf16 = jnp.bfloat16
    bt = xb_ref.shape[-1]

    wt1 = wt1_ref[...]
    wt2 = wt2_ref[...]

    for p in range(14):
        d = []
        for r in (2 * p, 2 * p + 1):
            strip = jnp.reshape(xb_ref[:, r:r + 5, :, :], (480, bt))
            d.append(jnp.dot(wt1, strip, preferred_element_type=f32))
        m4 = jnp.maximum(jnp.maximum(d[0][0:96, :], d[0][96:192, :]),
                         jnp.maximum(d[1][0:96, :], d[1][96:192, :]))
        pooled = jnp.maximum(m4 + b1_ref[...], 0.0)
        p1_ref[:, p, :, :] = jnp.reshape(pooled, (6, 16, bt)).astype(bf16)

    for p in range(5):
        d = []
        for r in (2 * p, 2 * p + 1):
            strip = jnp.reshape(p1_ref[:, r:r + 5, :, :], (480, bt))
            d.append(jnp.dot(wt2, strip, preferred_element_type=f32))
        m4 = jnp.maximum(jnp.maximum(d[0][0:128, :], d[0][128:256, :]),
                         jnp.maximum(d[1][0:128, :], d[1][128:256, :]))
        pooled = jnp.maximum(m4 + b2_ref[...], 0.0)
        act_ref[:, pl.ds(8 * p, 8), :] = jnp.reshape(pooled, (16, 8, bt))

    a = jnp.reshape(act_ref[...], (640, bt)).astype(bf16)
    y = jnp.dot(f1w_ref[...], a, preferred_element_type=f32)
    y = jnp.maximum(y + f1b_ref[...], 0.0).astype(bf16)
    y = jnp.dot(f2w_ref[...], y, preferred_element_type=f32)
    y = jnp.maximum(y + f2b_ref[...], 0.0).astype(bf16)
    y = jnp.dot(f3w_ref[...], y, preferred_element_type=f32)
    out_ref[...] = y + f3b_ref[...]


def kernel(conv1_w, conv1_b, conv2_w, conv2_b,
           fc1_w, fc1_b, fc2_w, fc2_b, fc3_w, fc3_b, x):
    B = x.shape[0]
    bt = _BT
    n_tiles = pl.cdiv(B, bt)
    bp = n_tiles * bt
    f32 = jnp.float32
    bf16 = jnp.bfloat16

    xb = jnp.transpose(x.astype(bf16), (1, 2, 3, 0))
    if bp != B:
        xb = jnp.pad(xb, ((0, 0), (0, 0), (0, 0), (0, bp - B)))

    w1f = jnp.concatenate([conv1_w.reshape(-1).astype(f32), jnp.zeros((1,), f32)])
    wt1 = jnp.take(w1f, _IDX1.reshape(-1)).reshape(192, 480).astype(bf16)
    w2f = jnp.concatenate([conv2_w.reshape(-1).astype(f32), jnp.zeros((1,), f32)])
    wt2 = jnp.take(w2f, _IDX2.reshape(-1)).reshape(256, 480).astype(bf16)
    b1 = (conv1_b.astype(f32)[:, None] * jnp.asarray(_T1MASK)).reshape(96, 1)
    b2 = (conv2_b.astype(f32)[:, None] * jnp.asarray(_T2MASK)).reshape(128, 1)

    def rep(shape):
        return pl.BlockSpec(shape, lambda t: (0,) * len(shape))

    out = pl.pallas_call(
        _lenet_kernel,
        out_shape=jax.ShapeDtypeStruct((10, bp), f32),
        grid=(n_tiles,),
        in_specs=[
            pl.BlockSpec((3, 32, 32, bt), lambda t: (0, 0, 0, t)),
            rep((192, 480)), rep((96, 1)),
            rep((256, 480)), rep((128, 1)),
            rep((120, 640)), rep((120, 1)),
            rep((84, 120)), rep((84, 1)),
            rep((10, 84)), rep((10, 1)),
        ],
        out_specs=pl.BlockSpec((10, bt), lambda t: (0, t)),
        scratch_shapes=[
            pltpu.VMEM((6, 14, 16, bt), bf16),
            pltpu.VMEM((16, 40, bt), f32),
        ],
        compiler_params=pltpu.CompilerParams(
            dimension_semantics=("parallel",)),
    )(xb, wt1, b1, wt2, b2,
      fc1_w.astype(bf16), fc1_b.astype(f32),
      fc2_w.astype(bf16), fc2_b.astype(f32),
      fc3_w.astype(bf16), fc3_b.astype(f32))
    return out[:, :B].T


# speedup vs baseline: 28.5375x; 28.5375x over previous
import numpy as np
import jax
import jax.numpy as jnp
from jax.experimental import pallas as pl
from jax.experimental.pallas import tpu as pltpu

_BT = 256

_T1MASK = (np.arange(16) < 14).astype(np.float32)
_T2MASK = (np.arange(8) < 5).astype(np.float32)


def _banded(w4, n_t, width, t_pad):
    co, ci = w4.shape[0], w4.shape[1]
    v = jnp.broadcast_to(w4[:, :, :, None, :], (co, ci, 5, n_t, 5))
    blocks = []
    for par in range(2):
        b = jnp.pad(v, ((0, 0), (0, 0), (0, 0), (0, 0),
                        (par, width + 2 - 5 - par)))
        flat = b.reshape(co, ci, 5, n_t * (width + 2))
        a = flat[..., :n_t * width].reshape(co, ci, 5, n_t, width)
        blocks.append(a)
    a = jnp.stack(blocks, 0)
    a = jnp.pad(a, ((0, 0), (0, 0), (0, 0), (0, 0), (0, t_pad - n_t), (0, 0)))
    return jnp.transpose(a, (0, 1, 4, 2, 3, 5))


def _lenet_kernel(xb_ref, wt1_ref, b1_ref, wt2_ref, b2_ref,
                  f1w_ref, f1b_ref, f2w_ref, f2b_ref, f3w_ref, f3b_ref,
                  out_ref,
                  p1_ref, act_ref):
    f32 = jnp.float32
    bf16 = jnp.bfloat16
    bt = xb_ref.shape[-1]

    wt1 = wt1_ref[...]
    wt2 = wt2_ref[...]

    for p in range(14):
        d = []
        for r in (2 * p, 2 * p + 1):
            strip = jnp.reshape(xb_ref[:, r:r + 5, :, :], (480, bt))
            d.append(jnp.dot(wt1, strip, preferred_element_type=f32))
        m4 = jnp.maximum(jnp.maximum(d[0][0:96, :], d[0][96:192, :]),
                         jnp.maximum(d[1][0:96, :], d[1][96:192, :]))
        pooled = jnp.maximum(m4 + b1_ref[...], 0.0)
        p1_ref[:, p, :, :] = jnp.reshape(pooled, (6, 16, bt)).astype(bf16)

    for p in range(5):
        d = []
        for r in (2 * p, 2 * p + 1):
            strip = jnp.reshape(p1_ref[:, r:r + 5, :, :], (480, bt))
            d.append(jnp.dot(wt2, strip, preferred_element_type=f32))
        m4 = jnp.maximum(jnp.maximum(d[0][0:128, :], d[0][128:256, :]),
                         jnp.maximum(d[1][0:128, :], d[1][128:256, :]))
        pooled = jnp.maximum(m4 + b2_ref[...], 0.0)
        act_ref[:, pl.ds(8 * p, 8), :] = jnp.reshape(pooled, (16, 8, bt))

    a = jnp.reshape(act_ref[...], (640, bt)).astype(bf16)
    y = jnp.dot(f1w_ref[...], a, preferred_element_type=f32)
    y = jnp.maximum(y + f1b_ref[...], 0.0).astype(bf16)
    y = jnp.dot(f2w_ref[...], y, preferred_element_type=f32)
    y = jnp.maximum(y + f2b_ref[...], 0.0).astype(bf16)
    y = jnp.dot(f3w_ref[...], y, preferred_element_type=f32)
    out_ref[...] = y + f3b_ref[...]


def kernel(conv1_w, conv1_b, conv2_w, conv2_b,
           fc1_w, fc1_b, fc2_w, fc2_b, fc3_w, fc3_b, x):
    B = x.shape[0]
    bt = _BT
    n_tiles = pl.cdiv(B, bt)
    bp = n_tiles * bt
    f32 = jnp.float32
    bf16 = jnp.bfloat16

    xb = jnp.transpose(x.astype(bf16), (1, 2, 3, 0))
    if bp != B:
        xb = jnp.pad(xb, ((0, 0), (0, 0), (0, 0), (0, bp - B)))

    wt1 = _banded(conv1_w.astype(f32).reshape(6, 3, 5, 5),
                  14, 32, 16).reshape(192, 480).astype(bf16)
    wt2 = _banded(conv2_w.astype(f32).reshape(16, 6, 5, 5),
                  5, 16, 8).reshape(256, 480).astype(bf16)
    b1 = (conv1_b.astype(f32)[:, None] * jnp.asarray(_T1MASK)).reshape(96, 1)
    b2 = (conv2_b.astype(f32)[:, None] * jnp.asarray(_T2MASK)).reshape(128, 1)

    def rep(shape):
        return pl.BlockSpec(shape, lambda t: (0,) * len(shape))

    out = pl.pallas_call(
        _lenet_kernel,
        out_shape=jax.ShapeDtypeStruct((10, bp), f32),
        grid=(n_tiles,),
        in_specs=[
            pl.BlockSpec((3, 32, 32, bt), lambda t: (0, 0, 0, t)),
            rep((192, 480)), rep((96, 1)),
            rep((256, 480)), rep((128, 1)),
            rep((120, 640)), rep((120, 1)),
            rep((84, 120)), rep((84, 1)),
            rep((10, 84)), rep((10, 1)),
        ],
        out_specs=pl.BlockSpec((10, bt), lambda t: (0, t)),
        scratch_shapes=[
            pltpu.VMEM((6, 14, 16, bt), bf16),
            pltpu.VMEM((16, 40, bt), f32),
        ],
        compiler_params=pltpu.CompilerParams(
            dimension_semantics=("parallel",)),
    )(xb, wt1, b1, wt2, b2,
      fc1_w.astype(bf16), fc1_b.astype(f32),
      fc2_w.astype(bf16), fc2_b.astype(f32),
      fc3_w.astype(bf16), fc3_b.astype(f32))
    return out[:, :B].T
